# baseline (speedup 1.0000x reference)
import functools

import jax
import jax.numpy as jnp
from jax import lax
from jax.experimental import pallas as pl
from jax.experimental.pallas import tpu as pltpu
from jax.experimental.pallas import tpu_sc as plsc

F32 = jnp.float32

NP = 100352
BLK = 1024
GRID = NP // BLK
EP = 32 * NP
EROWS = EP // 128
ROWS_PER_TILE = (EP // 32) // 128
CHUNK_ROWS = 4
N_CHUNKS = ROWS_PER_TILE // CHUNK_ROWS
SLICE = NP // 16

_mesh = plsc.VectorSubcoreMesh(core_axis_name="c", subcore_axis_name="s")
_sc_params = pltpu.CompilerParams(use_tc_tiling_on_sc=False)


@functools.partial(
    pl.kernel,
    mesh=_mesh,
    out_type=jax.ShapeDtypeStruct((2, NP), F32),
    compiler_params=_sc_params,
    scratch_types=[
        pltpu.VMEM((4 * CHUNK_ROWS, 128), jnp.int32),
        pltpu.VMEM((128,), F32),
        pltpu.VMEM_SHARED((NP,), F32),
        pltpu.SemaphoreType.DMA,
        pltpu.SemaphoreType.DMA,
    ],
)
def _deg_pass(col2d, zeros1, out, cidx, ones_v, acc, ssem, isem):
    c = lax.axis_index("c")
    s = lax.axis_index("s")
    wid = c * 16 + s
    pltpu.sync_copy(zeros1.at[pl.ds(s * SLICE, SLICE)],
                    acc.at[pl.ds(s * SLICE, SLICE)])
    for j in range(8):
        ones_v[pl.ds(j * 16, 16)] = jnp.ones((16,), F32)
    plsc.subcore_barrier()

    base = wid * ROWS_PER_TILE
    last = N_CHUNKS - 1

    def il_desc(t):
        q = t & 3
        return pltpu.make_async_copy(
            col2d.at[pl.ds(base + t * CHUNK_ROWS, CHUNK_ROWS)],
            cidx.at[pl.ds(q * CHUNK_ROWS, CHUNK_ROWS)], isem)

    def s_descs(t):
        q = t & 3
        return [pltpu.make_async_copy(ones_v, acc.at[cidx.at[q * CHUNK_ROWS + j]],
                                      ssem)
                for j in range(CHUNK_ROWS)]

    def s_fire(t):
        q = t & 3
        for j in range(CHUNK_ROWS):
            pltpu.async_copy(ones_v, acc.at[cidx.at[q * CHUNK_ROWS + j]],
                             ssem, add=True)

    for t in range(4):
        il_desc(t).start()
    for t in range(3):
        il_desc(t).wait()
        s_fire(t)

    def body(t, carry):
        for d in s_descs(t - 3):
            d.wait()
        il_desc(t).wait()
        il_desc(jnp.minimum(t + 1, last)).start()
        s_fire(t)
        return carry

    lax.fori_loop(3, N_CHUNKS, body, 0)
    for t in (N_CHUNKS - 3, N_CHUNKS - 2, N_CHUNKS - 1):
        for d in s_descs(t):
            d.wait()
    il_desc(last).wait()

    plsc.subcore_barrier()
    pltpu.sync_copy(acc.at[pl.ds(s * SLICE, SLICE)],
                    out.at[c, pl.ds(s * SLICE, SLICE)])


@functools.partial(
    pl.kernel,
    mesh=_mesh,
    out_type=jax.ShapeDtypeStruct((2, NP, 16), F32),
    compiler_params=_sc_params,
    scratch_types=[
        pltpu.VMEM((4 * CHUNK_ROWS * 128,), jnp.int32),
        pltpu.VMEM((4 * CHUNK_ROWS, 128), jnp.int32),
        pltpu.VMEM((2 * CHUNK_ROWS * 128, 16), F32),
        pltpu.VMEM_SHARED((NP, 16), F32),
        pltpu.SemaphoreType.DMA,
        pltpu.SemaphoreType.DMA,
        pltpu.SemaphoreType.DMA,
    ],
)
def _edge_pass(g2d, row1d, col2d, zeros2, out, ridx, cidx, rows_v, acc,
               gsem, ssem, isem):
    c = lax.axis_index("c")
    s = lax.axis_index("s")
    wid = c * 16 + s
    pltpu.sync_copy(zeros2.at[pl.ds(s * SLICE, SLICE)],
                    acc.at[pl.ds(s * SLICE, SLICE)])
    plsc.subcore_barrier()

    base = wid * ROWS_PER_TILE
    base_e = wid * (ROWS_PER_TILE * 128)
    gsz = CHUNK_ROWS * 128
    last = N_CHUNKS - 1

    def il_descs(t):
        q = t & 3
        return [pltpu.make_async_copy(
                    row1d.at[pl.ds(base_e + t * gsz, gsz)],
                    ridx.at[pl.ds(q * gsz, gsz)], isem),
                pltpu.make_async_copy(
                    col2d.at[pl.ds(base + t * CHUNK_ROWS, CHUNK_ROWS)],
                    cidx.at[pl.ds(q * CHUNK_ROWS, CHUNK_ROWS)], isem)]

    def g_descs(t):
        q = t & 3
        p = t & 1
        return [pltpu.make_async_copy(
                    g2d.at[ridx.at[pl.ds(q * gsz, gsz)]],
                    rows_v.at[pl.ds(p * gsz, gsz)], gsem)]

    def s_descs(t):
        q = t & 3
        p = t & 1
        return [pltpu.make_async_copy(
                    rows_v.at[pl.ds((p * CHUNK_ROWS + j) * 128, 128)],
                    acc.at[cidx.at[q * CHUNK_ROWS + j]], ssem)
                for j in range(CHUNK_ROWS)]

    def fire(descs, add=False):
        for d in descs:
            d.start(add=add)

    def drain(descs):
        for d in descs:
            d.wait()

    fire(il_descs(0))
    fire(il_descs(1))
    fire(il_descs(2))
    drain(il_descs(0))
    fire(g_descs(0))
    drain(il_descs(1))
    fire(g_descs(1))
    drain(g_descs(0))
    fire(s_descs(0), add=True)

    def body(t, carry):
        drain(g_descs(t))
        drain(s_descs(t - 1))
        drain(il_descs(t + 1))
        fire(il_descs(jnp.minimum(t + 2, last)))
        fire(g_descs(t + 1))
        fire(s_descs(t), add=True)
        return carry

    lax.fori_loop(1, N_CHUNKS - 1, body, 0)

    drain(g_descs(last))
    drain(s_descs(last - 1))
    fire(s_descs(last), add=True)
    drain(s_descs(last))
    drain(il_descs(last))

    plsc.subcore_barrier()
    pltpu.sync_copy(acc.at[pl.ds(s * SLICE, SLICE)],
                    out.at[c, pl.ds(s * SLICE, SLICE)])


_SELU_SCALE = 1.0507009873554805
_SELU_ALPHA = 1.6732632423543772


def _selu(x):
    return _SELU_SCALE * jnp.where(x > 0, x, _SELU_ALPHA * (jnp.exp(x) - 1.0))


def _k0_body(degp, xr, w1, b1, wc1, dis_o, hw1_o, g1_o):
    deg = degp[0, :] + degp[1, :] + 1.0
    dis = lax.rsqrt(deg)
    dis2 = jnp.broadcast_to(dis[:, None], (BLK, 16))
    h0 = jnp.dot(xr[...], w1[...], preferred_element_type=F32) + b1[...]
    hw1 = jnp.dot(h0, wc1[...], preferred_element_type=F32)
    dis_o[...] = dis2
    hw1_o[...] = hw1
    g1_o[...] = dis2 * hw1


def _k1_body(raw, dis, hw1, bc1, wc2, hw2_o, g2_o):
    d = dis[...]
    out1 = d * (raw[0] + raw[1]) + d * d * hw1[...] + bc1[...]
    h1 = _selu(out1)
    hw2 = jnp.dot(h1, wc2[...], preferred_element_type=F32)
    hw2_o[...] = hw2
    g2_o[...] = d * hw2


def _k2_body(raw, dis, hw2, bc2, w2, b2, out_o):
    d = dis[...]
    out2 = d * (raw[0] + raw[1]) + d * d * hw2[...] + bc2[...]
    h2 = _selu(out2)
    z = jnp.dot(h2, w2[...], preferred_element_type=F32) + b2[...]
    m = jnp.max(z, axis=1, keepdims=True)
    lse = m + jnp.log(jnp.sum(jnp.exp(z - m), axis=1, keepdims=True))
    out_o[...] = z - lse


def _full(shape):
    return pl.BlockSpec(shape, lambda i: (0,) * len(shape))


def _rows(f):
    return pl.BlockSpec((BLK, f), lambda i: (i, 0))


_k0 = pl.pallas_call(
    _k0_body,
    grid=(GRID,),
    in_specs=[pl.BlockSpec((2, BLK), lambda i: (0, i)),
              _rows(5), _full((5, 16)), _full((1, 16)), _full((16, 16))],
    out_specs=[_rows(16), _rows(16), _rows(16)],
    out_shape=[jax.ShapeDtypeStruct((NP, 16), F32)] * 3,
)

_k1 = pl.pallas_call(
    _k1_body,
    grid=(GRID,),
    in_specs=[pl.BlockSpec((2, BLK, 16), lambda i: (0, i, 0)),
              _rows(16), _rows(16), _full((1, 16)), _full((16, 16))],
    out_specs=[_rows(16), _rows(16)],
    out_shape=[jax.ShapeDtypeStruct((NP, 16), F32)] * 2,
)

_k2 = pl.pallas_call(
    _k2_body,
    grid=(GRID,),
    in_specs=[pl.BlockSpec((2, BLK, 16), lambda i: (0, i, 0)),
              _rows(16), _rows(16), _full((1, 16)), _full((16, 2)),
              _full((1, 2))],
    out_specs=_rows(2),
    out_shape=jax.ShapeDtypeStruct((NP, 2), F32),
)


def kernel(x, edge_index, W1, b1, Wc1, bc1, Wc2, bc2, W2, b2):
    n = x.shape[0]
    e = edge_index.shape[1]
    pad = jnp.full((1, EP - e), NP - 1, dtype=edge_index.dtype)
    eip = jnp.concatenate([edge_index, jnp.concatenate([pad, pad], axis=0)],
                          axis=1)
    row1d = eip[0]
    col2d = eip[1].reshape(EROWS, 128)
    xp = jnp.zeros((NP, x.shape[1]), F32).at[:n].set(x)
    zeros1 = jnp.zeros((NP,), F32)
    zeros2 = jnp.zeros((NP, 16), F32)

    deg_p = _deg_pass(col2d, zeros1)
    dis16, hw1, g1 = _k0(deg_p, xp, W1, b1.reshape(1, 16), Wc1)
    raw1 = _edge_pass(g1, row1d, col2d, zeros2)
    hw2, g2 = _k1(raw1, dis16, hw1, bc1.reshape(1, 16), Wc2)
    raw2 = _edge_pass(g2, row1d, col2d, zeros2)
    outp = _k2(raw2, dis16, hw2, bc2.reshape(1, 16), W2, b2.reshape(1, 2))
    return outp[:n]


# speedup vs baseline: 1.4691x; 1.4691x over previous
import functools

import jax
import jax.numpy as jnp
from jax import lax
from jax.experimental import pallas as pl
from jax.experimental.pallas import tpu as pltpu
from jax.experimental.pallas import tpu_sc as plsc

F32 = jnp.float32

NP = 100352
BLK = 1024
GRID = NP // BLK
EP = 32 * NP
EROWS = EP // 128
ROWS_PER_TILE = (EP // 32) // 128
CHUNK_ROWS = 4
N_CHUNKS = ROWS_PER_TILE // CHUNK_ROWS
SLICE = NP // 16

_mesh = plsc.VectorSubcoreMesh(core_axis_name="c", subcore_axis_name="s")
_sc_params = pltpu.CompilerParams(use_tc_tiling_on_sc=False)


@functools.partial(
    pl.kernel,
    mesh=_mesh,
    out_type=jax.ShapeDtypeStruct((2, NP), F32),
    compiler_params=_sc_params,
    scratch_types=[
        pltpu.VMEM((4 * CHUNK_ROWS, 128), jnp.int32),
        pltpu.VMEM((128,), F32),
        pltpu.VMEM_SHARED((NP,), F32),
        pltpu.SemaphoreType.DMA,
        pltpu.SemaphoreType.DMA,
    ],
)
def _deg_pass(col2d, zeros1, out, cidx, ones_v, acc, ssem, isem):
    c = lax.axis_index("c")
    s = lax.axis_index("s")
    wid = c * 16 + s
    pltpu.sync_copy(zeros1.at[pl.ds(s * SLICE, SLICE)],
                    acc.at[pl.ds(s * SLICE, SLICE)])
    for j in range(8):
        ones_v[pl.ds(j * 16, 16)] = jnp.ones((16,), F32)
    plsc.subcore_barrier()

    base = wid * ROWS_PER_TILE
    last = N_CHUNKS - 1

    def il_desc(t):
        q = t & 3
        return pltpu.make_async_copy(
            col2d.at[pl.ds(base + t * CHUNK_ROWS, CHUNK_ROWS)],
            cidx.at[pl.ds(q * CHUNK_ROWS, CHUNK_ROWS)], isem)

    def s_descs(t):
        q = t & 3
        return [pltpu.make_async_copy(ones_v, acc.at[cidx.at[q * CHUNK_ROWS + j]],
                                      ssem)
                for j in range(CHUNK_ROWS)]

    def s_fire(t):
        q = t & 3
        for j in range(CHUNK_ROWS):
            pltpu.async_copy(ones_v, acc.at[cidx.at[q * CHUNK_ROWS + j]],
                             ssem, add=True)

    for t in range(4):
        il_desc(t).start()
    for t in range(3):
        il_desc(t).wait()
        s_fire(t)

    def body(t, carry):
        for d in s_descs(t - 3):
            d.wait()
        il_desc(t).wait()
        il_desc(jnp.minimum(t + 1, last)).start()
        s_fire(t)
        return carry

    lax.fori_loop(3, N_CHUNKS, body, 0)
    for t in (N_CHUNKS - 3, N_CHUNKS - 2, N_CHUNKS - 1):
        for d in s_descs(t):
            d.wait()
    il_desc(last).wait()

    plsc.subcore_barrier()
    pltpu.sync_copy(acc.at[pl.ds(s * SLICE, SLICE)],
                    out.at[c, pl.ds(s * SLICE, SLICE)])


@functools.partial(
    pl.kernel,
    mesh=_mesh,
    out_type=jax.ShapeDtypeStruct((2, NP, 16), F32),
    compiler_params=_sc_params,
    scratch_types=[
        pltpu.VMEM((4 * CHUNK_ROWS * 128,), jnp.int32),
        pltpu.VMEM((4 * CHUNK_ROWS, 128), jnp.int32),
        pltpu.VMEM((2 * CHUNK_ROWS * 128, 16), F32),
        pltpu.VMEM_SHARED((NP, 16), F32),
        pltpu.SemaphoreType.DMA,
        pltpu.SemaphoreType.DMA,
        pltpu.SemaphoreType.DMA,
    ],
)
def _edge_pass(g2d, row1d, col2d, zeros2, out, ridx, cidx, rows_v, acc,
               gsem, ssem, isem):
    c = lax.axis_index("c")
    s = lax.axis_index("s")
    wid = c * 16 + s
    pltpu.sync_copy(zeros2.at[pl.ds(s * SLICE, SLICE)],
                    acc.at[pl.ds(s * SLICE, SLICE)])
    plsc.subcore_barrier()

    base = wid * ROWS_PER_TILE
    base_e = wid * (ROWS_PER_TILE * 128)
    gsz = CHUNK_ROWS * 128
    last = N_CHUNKS - 1

    def il_descs(t):
        q = t & 3
        return [pltpu.make_async_copy(
                    row1d.at[pl.ds(base_e + t * gsz, gsz)],
                    ridx.at[pl.ds(q * gsz, gsz)], isem),
                pltpu.make_async_copy(
                    col2d.at[pl.ds(base + t * CHUNK_ROWS, CHUNK_ROWS)],
                    cidx.at[pl.ds(q * CHUNK_ROWS, CHUNK_ROWS)], isem)]

    def g_descs(t):
        q = t & 3
        p = t & 1
        return [pltpu.make_async_copy(
                    g2d.at[ridx.at[pl.ds(q * gsz, gsz)]],
                    rows_v.at[pl.ds(p * gsz, gsz)], gsem)]

    def s_descs(t):
        q = t & 3
        p = t & 1
        return [pltpu.make_async_copy(
                    rows_v.at[pl.ds((p * CHUNK_ROWS + j) * 128, 128)],
                    acc.at[cidx.at[q * CHUNK_ROWS + j]], ssem)
                for j in range(CHUNK_ROWS)]

    def fire(descs, add=False):
        for d in descs:
            d.start(add=add)

    def drain(descs):
        for d in descs:
            d.wait()

    fire(il_descs(0))
    fire(il_descs(1))
    fire(il_descs(2))
    drain(il_descs(0))
    fire(g_descs(0))
    drain(il_descs(1))
    fire(g_descs(1))
    drain(g_descs(0))
    fire(s_descs(0), add=True)

    def body(t, carry):
        drain(g_descs(t))
        drain(s_descs(t - 1))
        drain(il_descs(t + 1))
        fire(il_descs(jnp.minimum(t + 2, last)))
        fire(g_descs(t + 1))
        fire(s_descs(t), add=True)
        return carry

    lax.fori_loop(1, N_CHUNKS - 1, body, 0)

    drain(g_descs(last))
    drain(s_descs(last - 1))
    fire(s_descs(last), add=True)
    drain(s_descs(last))
    drain(il_descs(last))

    plsc.subcore_barrier()
    pltpu.sync_copy(acc.at[pl.ds(s * SLICE, SLICE)],
                    out.at[c, pl.ds(s * SLICE, SLICE)])


_SELU_SCALE = 1.0507009873554805
_SELU_ALPHA = 1.6732632423543772


def _selu(x):
    return _SELU_SCALE * jnp.where(x > 0, x, _SELU_ALPHA * (jnp.exp(x) - 1.0))


NP8 = NP // 8
BR = NP8 // 8


def _dis128(degp, sel):
    deg8 = degp[0] + degp[1] + 1.0
    return jnp.dot(lax.rsqrt(deg8), sel[...], preferred_element_type=F32)


def _k0_body(degp, sel, xr, w1, b1, wc1, hw1_o, g1_o):
    d = _dis128(degp, sel)
    h0 = jnp.dot(xr[...], w1[...], preferred_element_type=F32) + b1[...]
    hw1 = jnp.dot(h0, wc1[...], preferred_element_type=F32)
    hw1_o[...] = hw1
    g1_o[...] = d * hw1


def _k1_body(degp, sel, raw, hw1, bc1, wc2, hw2_o, g2_o):
    d = _dis128(degp, sel)
    out1 = d * (raw[0] + raw[1]) + d * d * hw1[...] + bc1[...]
    h1 = _selu(out1)
    hw2 = jnp.dot(h1, wc2[...], preferred_element_type=F32)
    hw2_o[...] = hw2
    g2_o[...] = d * hw2


def _k2_body(degp, sel, raw, hw2, bc2, w2, b2, pswap, out_o):
    d = _dis128(degp, sel)
    out2 = d * (raw[0] + raw[1]) + d * d * hw2[...] + bc2[...]
    h2 = _selu(out2)
    z = jnp.dot(h2, w2[...], preferred_element_type=F32) + b2[...]
    zs = jnp.dot(z, pswap[...], preferred_element_type=F32)
    m = jnp.maximum(z, zs)
    lse = m + jnp.log(jnp.exp(z - m) + jnp.exp(zs - m))
    out_o[...] = z - lse


def _full(shape):
    return pl.BlockSpec(shape, lambda i: (0,) * len(shape))


def _rows(f):
    return pl.BlockSpec((BR, f), lambda i: (i, 0))


_DEGSPEC = pl.BlockSpec((2, BR, 8), lambda i: (0, i, 0))

_k0 = pl.pallas_call(
    _k0_body,
    grid=(8,),
    in_specs=[_DEGSPEC, _full((8, 128)), _rows(40), _full((40, 128)),
              _full((1, 128)), _full((128, 128))],
    out_specs=[_rows(128), _rows(128)],
    out_shape=[jax.ShapeDtypeStruct((NP8, 128), F32)] * 2,
)

_k1 = pl.pallas_call(
    _k1_body,
    grid=(8,),
    in_specs=[_DEGSPEC, _full((8, 128)),
              pl.BlockSpec((2, BR, 128), lambda i: (0, i, 0)),
              _rows(128), _full((1, 128)), _full((128, 128))],
    out_specs=[_rows(128), _rows(128)],
    out_shape=[jax.ShapeDtypeStruct((NP8, 128), F32)] * 2,
)

_k2 = pl.pallas_call(
    _k2_body,
    grid=(8,),
    in_specs=[_DEGSPEC, _full((8, 128)),
              pl.BlockSpec((2, BR, 128), lambda i: (0, i, 0)),
              _rows(128), _full((1, 128)), _full((128, 16)),
              _full((1, 16)), _full((16, 16))],
    out_specs=_rows(16),
    out_shape=jax.ShapeDtypeStruct((NP8, 16), F32),
)


def kernel(x, edge_index, W1, b1, Wc1, bc1, Wc2, bc2, W2, b2):
    n = x.shape[0]
    e = edge_index.shape[1]
    pad = jnp.full((1, EP - e), NP - 1, dtype=edge_index.dtype)
    eip = jnp.concatenate([edge_index, jnp.concatenate([pad, pad], axis=0)],
                          axis=1)
    row1d = eip[0]
    col2d = eip[1].reshape(EROWS, 128)
    xp = jnp.zeros((NP, x.shape[1]), F32).at[:n].set(x)
    x8 = xp.reshape(NP8, 40)
    zeros1 = jnp.zeros((NP,), F32)
    zeros2 = jnp.zeros((NP, 16), F32)

    eye8 = jnp.eye(8, dtype=F32)
    sel = jnp.kron(eye8, jnp.ones((1, 16), F32))
    w1b = jnp.kron(eye8, W1)
    wc1b = jnp.kron(eye8, Wc1)
    wc2b = jnp.kron(eye8, Wc2)
    w2b = jnp.kron(eye8, W2)
    pswap = jnp.kron(eye8, jnp.array([[0., 1.], [1., 0.]], F32))
    b1t = jnp.tile(b1, 8).reshape(1, 128)
    bc1t = jnp.tile(bc1, 8).reshape(1, 128)
    bc2t = jnp.tile(bc2, 8).reshape(1, 128)
    b2t = jnp.tile(b2, 8).reshape(1, 16)

    deg_p = _deg_pass(col2d, zeros1)
    deg8 = deg_p.reshape(2, NP8, 8)
    hw1, g1 = _k0(deg8, sel, x8, w1b, b1t, wc1b)
    raw1 = _edge_pass(g1.reshape(NP, 16), row1d, col2d, zeros2)
    hw2, g2 = _k1(deg8, sel, raw1.reshape(2, NP8, 128), hw1, bc1t, wc2b)
    raw2 = _edge_pass(g2.reshape(NP, 16), row1d, col2d, zeros2)
    out8 = _k2(deg8, sel, raw2.reshape(2, NP8, 128), hw2, bc2t, w2b, b2t,
               pswap)
    return out8.reshape(NP, 2)[:n]


# speedup vs baseline: 1.4734x; 1.0029x over previous
import functools

import jax
import jax.numpy as jnp
from jax import lax
from jax.experimental import pallas as pl
from jax.experimental.pallas import tpu as pltpu
from jax.experimental.pallas import tpu_sc as plsc

F32 = jnp.float32

NP = 100352
BLK = 1024
GRID = NP // BLK
EP = 32 * NP
EROWS = EP // 128
ROWS_PER_TILE = (EP // 32) // 128
CHUNK_ROWS = 4
N_CHUNKS = ROWS_PER_TILE // CHUNK_ROWS
SLICE = NP // 16

_mesh = plsc.VectorSubcoreMesh(core_axis_name="c", subcore_axis_name="s")
_sc_params = pltpu.CompilerParams(use_tc_tiling_on_sc=False)


@functools.partial(
    pl.kernel,
    mesh=_mesh,
    out_type=jax.ShapeDtypeStruct((2, NP), F32),
    compiler_params=_sc_params,
    scratch_types=[
        pltpu.VMEM((4 * CHUNK_ROWS * 128,), jnp.int32),
        pltpu.VMEM((CHUNK_ROWS * 128,), F32),
        pltpu.VMEM_SHARED((NP,), F32),
        pltpu.SemaphoreType.DMA,
        pltpu.SemaphoreType.DMA,
    ],
)
def _deg_pass(col1d, zeros1, out, cidx, ones_v, acc, ssem, isem):
    c = lax.axis_index("c")
    s = lax.axis_index("s")
    wid = c * 16 + s
    pltpu.sync_copy(zeros1.at[pl.ds(s * SLICE, SLICE)],
                    acc.at[pl.ds(s * SLICE, SLICE)])

    gsz = CHUNK_ROWS * 128

    def fill_ones(j, carry):
        ones_v[pl.ds(j * 16, 16)] = jnp.ones((16,), F32)
        return carry

    lax.fori_loop(0, gsz // 16, fill_ones, 0)
    plsc.subcore_barrier()

    base_e = wid * (ROWS_PER_TILE * 128)
    last = N_CHUNKS - 1

    def il_desc(t):
        q = t & 3
        return pltpu.make_async_copy(
            col1d.at[pl.ds(base_e + t * gsz, gsz)],
            cidx.at[pl.ds(q * gsz, gsz)], isem)

    def s_descs(t):
        q = t & 3
        return [pltpu.make_async_copy(ones_v,
                                      acc.at[cidx.at[pl.ds(q * gsz, gsz)]],
                                      ssem)]

    def s_fire(t):
        for d in s_descs(t):
            d.start(add=True)

    for t in range(4):
        il_desc(t).start()
    for t in range(3):
        il_desc(t).wait()
        s_fire(t)

    def body(t, carry):
        for d in s_descs(t - 3):
            d.wait()
        il_desc(t).wait()
        il_desc(jnp.minimum(t + 1, last)).start()
        s_fire(t)
        return carry

    lax.fori_loop(3, N_CHUNKS, body, 0)
    for t in (N_CHUNKS - 3, N_CHUNKS - 2, N_CHUNKS - 1):
        for d in s_descs(t):
            d.wait()
    il_desc(last).wait()

    plsc.subcore_barrier()
    pltpu.sync_copy(acc.at[pl.ds(s * SLICE, SLICE)],
                    out.at[c, pl.ds(s * SLICE, SLICE)])


@functools.partial(
    pl.kernel,
    mesh=_mesh,
    out_type=jax.ShapeDtypeStruct((2, NP, 16), F32),
    compiler_params=_sc_params,
    scratch_types=[
        pltpu.VMEM((4 * CHUNK_ROWS * 128,), jnp.int32),
        pltpu.VMEM((4 * CHUNK_ROWS * 128,), jnp.int32),
        pltpu.VMEM((2 * CHUNK_ROWS * 128, 16), F32),
        pltpu.VMEM_SHARED((NP, 16), F32),
        pltpu.SemaphoreType.DMA,
        pltpu.SemaphoreType.DMA,
        pltpu.SemaphoreType.DMA,
    ],
)
def _edge_pass(g2d, row1d, col1d, zeros2, out, ridx, cidx, rows_v, acc,
               gsem, ssem, isem):
    c = lax.axis_index("c")
    s = lax.axis_index("s")
    wid = c * 16 + s
    pltpu.sync_copy(zeros2.at[pl.ds(s * SLICE, SLICE)],
                    acc.at[pl.ds(s * SLICE, SLICE)])
    plsc.subcore_barrier()

    base_e = wid * (ROWS_PER_TILE * 128)
    gsz = CHUNK_ROWS * 128
    last = N_CHUNKS - 1

    def il_descs(t):
        q = t & 3
        return [pltpu.make_async_copy(
                    row1d.at[pl.ds(base_e + t * gsz, gsz)],
                    ridx.at[pl.ds(q * gsz, gsz)], isem),
                pltpu.make_async_copy(
                    col1d.at[pl.ds(base_e + t * gsz, gsz)],
                    cidx.at[pl.ds(q * gsz, gsz)], isem)]

    def g_descs(t):
        q = t & 3
        p = t & 1
        return [pltpu.make_async_copy(
                    g2d.at[ridx.at[pl.ds(q * gsz, gsz)]],
                    rows_v.at[pl.ds(p * gsz, gsz)], gsem)]

    def s_descs(t):
        q = t & 3
        p = t & 1
        return [pltpu.make_async_copy(
                    rows_v.at[pl.ds(p * gsz, gsz)],
                    acc.at[cidx.at[pl.ds(q * gsz, gsz)]], ssem)]

    def fire(descs, add=False):
        for d in descs:
            d.start(add=add)

    def drain(descs):
        for d in descs:
            d.wait()

    fire(il_descs(0))
    fire(il_descs(1))
    fire(il_descs(2))
    drain(il_descs(0))
    fire(g_descs(0))
    drain(il_descs(1))
    fire(g_descs(1))
    drain(g_descs(0))
    fire(s_descs(0), add=True)

    def body(t, carry):
        drain(g_descs(t))
        drain(s_descs(t - 1))
        drain(il_descs(t + 1))
        fire(il_descs(jnp.minimum(t + 2, last)))
        fire(g_descs(t + 1))
        fire(s_descs(t), add=True)
        return carry

    lax.fori_loop(1, N_CHUNKS - 1, body, 0)

    drain(g_descs(last))
    drain(s_descs(last - 1))
    fire(s_descs(last), add=True)
    drain(s_descs(last))
    drain(il_descs(last))

    plsc.subcore_barrier()
    pltpu.sync_copy(acc.at[pl.ds(s * SLICE, SLICE)],
                    out.at[c, pl.ds(s * SLICE, SLICE)])


_SELU_SCALE = 1.0507009873554805
_SELU_ALPHA = 1.6732632423543772


def _selu(x):
    return _SELU_SCALE * jnp.where(x > 0, x, _SELU_ALPHA * (jnp.exp(x) - 1.0))


NP8 = NP // 8
BR = NP8 // 8


def _dis128(degp, sel):
    deg8 = degp[0] + degp[1] + 1.0
    return jnp.dot(lax.rsqrt(deg8), sel[...], preferred_element_type=F32)


def _k0_body(degp, sel, xr, w1, b1, wc1, hw1_o, g1_o):
    d = _dis128(degp, sel)
    h0 = jnp.dot(xr[...], w1[...], preferred_element_type=F32) + b1[...]
    hw1 = jnp.dot(h0, wc1[...], preferred_element_type=F32)
    hw1_o[...] = hw1
    g1_o[...] = d * hw1


def _k1_body(degp, sel, raw, hw1, bc1, wc2, hw2_o, g2_o):
    d = _dis128(degp, sel)
    out1 = d * (raw[0] + raw[1]) + d * d * hw1[...] + bc1[...]
    h1 = _selu(out1)
    hw2 = jnp.dot(h1, wc2[...], preferred_element_type=F32)
    hw2_o[...] = hw2
    g2_o[...] = d * hw2


def _k2_body(degp, sel, raw, hw2, bc2, w2, b2, pswap, out_o):
    d = _dis128(degp, sel)
    out2 = d * (raw[0] + raw[1]) + d * d * hw2[...] + bc2[...]
    h2 = _selu(out2)
    z = jnp.dot(h2, w2[...], preferred_element_type=F32) + b2[...]
    zs = jnp.dot(z, pswap[...], preferred_element_type=F32)
    m = jnp.maximum(z, zs)
    lse = m + jnp.log(jnp.exp(z - m) + jnp.exp(zs - m))
    out_o[...] = z - lse


def _full(shape):
    return pl.BlockSpec(shape, lambda i: (0,) * len(shape))


def _rows(f):
    return pl.BlockSpec((BR, f), lambda i: (i, 0))


_DEGSPEC = pl.BlockSpec((2, BR, 8), lambda i: (0, i, 0))

_k0 = pl.pallas_call(
    _k0_body,
    grid=(8,),
    in_specs=[_DEGSPEC, _full((8, 128)), _rows(40), _full((40, 128)),
              _full((1, 128)), _full((128, 128))],
    out_specs=[_rows(128), _rows(128)],
    out_shape=[jax.ShapeDtypeStruct((NP8, 128), F32)] * 2,
)

_k1 = pl.pallas_call(
    _k1_body,
    grid=(8,),
    in_specs=[_DEGSPEC, _full((8, 128)),
              pl.BlockSpec((2, BR, 128), lambda i: (0, i, 0)),
              _rows(128), _full((1, 128)), _full((128, 128))],
    out_specs=[_rows(128), _rows(128)],
    out_shape=[jax.ShapeDtypeStruct((NP8, 128), F32)] * 2,
)

_k2 = pl.pallas_call(
    _k2_body,
    grid=(8,),
    in_specs=[_DEGSPEC, _full((8, 128)),
              pl.BlockSpec((2, BR, 128), lambda i: (0, i, 0)),
              _rows(128), _full((1, 128)), _full((128, 16)),
              _full((1, 16)), _full((16, 16))],
    out_specs=_rows(16),
    out_shape=jax.ShapeDtypeStruct((NP8, 16), F32),
)


def kernel(x, edge_index, W1, b1, Wc1, bc1, Wc2, bc2, W2, b2):
    n = x.shape[0]
    e = edge_index.shape[1]
    pad = jnp.full((1, EP - e), NP - 1, dtype=edge_index.dtype)
    eip = jnp.concatenate([edge_index, jnp.concatenate([pad, pad], axis=0)],
                          axis=1)
    row1d = eip[0]
    col1d = eip[1]
    xp = jnp.zeros((NP, x.shape[1]), F32).at[:n].set(x)
    x8 = xp.reshape(NP8, 40)
    zeros1 = jnp.zeros((NP,), F32)
    zeros2 = jnp.zeros((NP, 16), F32)

    eye8 = jnp.eye(8, dtype=F32)
    sel = jnp.kron(eye8, jnp.ones((1, 16), F32))
    w1b = jnp.kron(eye8, W1)
    wc1b = jnp.kron(eye8, Wc1)
    wc2b = jnp.kron(eye8, Wc2)
    w2b = jnp.kron(eye8, W2)
    pswap = jnp.kron(eye8, jnp.array([[0., 1.], [1., 0.]], F32))
    b1t = jnp.tile(b1, 8).reshape(1, 128)
    bc1t = jnp.tile(bc1, 8).reshape(1, 128)
    bc2t = jnp.tile(bc2, 8).reshape(1, 128)
    b2t = jnp.tile(b2, 8).reshape(1, 16)

    deg_p = _deg_pass(col1d, zeros1)
    deg8 = deg_p.reshape(2, NP8, 8)
    hw1, g1 = _k0(deg8, sel, x8, w1b, b1t, wc1b)
    raw1 = _edge_pass(g1.reshape(NP, 16), row1d, col1d, zeros2)
    hw2, g2 = _k1(deg8, sel, raw1.reshape(2, NP8, 128), hw1, bc1t, wc2b)
    raw2 = _edge_pass(g2.reshape(NP, 16), row1d, col1d, zeros2)
    out8 = _k2(deg8, sel, raw2.reshape(2, NP8, 128), hw2, bc2t, w2b, b2t,
               pswap)
    return out8.reshape(NP, 2)[:n]


# speedup vs baseline: 1.7257x; 1.1713x over previous
import functools

import jax
import jax.numpy as jnp
from jax import lax
from jax.experimental import pallas as pl
from jax.experimental.pallas import tpu as pltpu
from jax.experimental.pallas import tpu_sc as plsc

F32 = jnp.float32

NP = 100352
E_TOT = 3200000
EPT = E_TOT // 32
GSZ = 512
N_CHUNKS = EPT // GSZ
TAIL = EPT - N_CHUNKS * GSZ
SLICE = NP // 16

_mesh = plsc.VectorSubcoreMesh(core_axis_name="c", subcore_axis_name="s")
_sc_params = pltpu.CompilerParams(use_tc_tiling_on_sc=False)


@functools.partial(
    pl.kernel,
    mesh=_mesh,
    out_type=jax.ShapeDtypeStruct((2, NP), F32),
    compiler_params=_sc_params,
    scratch_types=[
        pltpu.VMEM((4 * GSZ,), jnp.int32),
        pltpu.VMEM((GSZ,), F32),
        pltpu.VMEM_SHARED((NP,), F32),
        pltpu.SemaphoreType.DMA,
        pltpu.SemaphoreType.DMA,
    ],
)
def _deg_pass(ei, zeros1, out, cidx, ones_v, acc, ssem, isem):
    c = lax.axis_index("c")
    s = lax.axis_index("s")
    wid = c * 16 + s
    pltpu.sync_copy(zeros1.at[pl.ds(s * SLICE, SLICE)],
                    acc.at[pl.ds(s * SLICE, SLICE)])

    def fill_ones(j, carry):
        ones_v[pl.ds(j * 16, 16)] = jnp.ones((16,), F32)
        return carry

    lax.fori_loop(0, GSZ // 16, fill_ones, 0)
    plsc.subcore_barrier()

    base_e = wid * EPT
    last = N_CHUNKS - 1

    def il_desc(t):
        q = t & 3
        return pltpu.make_async_copy(
            ei.at[1, pl.ds(base_e + t * GSZ, GSZ)],
            cidx.at[pl.ds(q * GSZ, GSZ)], isem)

    def s_descs(t):
        q = t & 3
        return [pltpu.make_async_copy(ones_v,
                                      acc.at[cidx.at[pl.ds(q * GSZ, GSZ)]],
                                      ssem)]

    def s_fire(t):
        for d in s_descs(t):
            d.start(add=True)

    for t in range(4):
        il_desc(t).start()
    for t in range(3):
        il_desc(t).wait()
        s_fire(t)

    def body(t, carry):
        for d in s_descs(t - 3):
            d.wait()
        il_desc(t).wait()
        il_desc(jnp.minimum(t + 1, last)).start()
        s_fire(t)
        return carry

    lax.fori_loop(3, N_CHUNKS, body, 0)
    for t in (N_CHUNKS - 3, N_CHUNKS - 2, N_CHUNKS - 1):
        for d in s_descs(t):
            d.wait()
    il_desc(last).wait()

    pltpu.sync_copy(ei.at[1, pl.ds(base_e + N_CHUNKS * GSZ, TAIL)],
                    cidx.at[pl.ds(0, TAIL)])
    tdesc = pltpu.make_async_copy(ones_v.at[pl.ds(0, TAIL)],
                                  acc.at[cidx.at[pl.ds(0, TAIL)]], ssem)
    tdesc.start(add=True)
    tdesc.wait()

    plsc.subcore_barrier()
    pltpu.sync_copy(acc.at[pl.ds(s * SLICE, SLICE)],
                    out.at[c, pl.ds(s * SLICE, SLICE)])


@functools.partial(
    pl.kernel,
    mesh=_mesh,
    out_type=jax.ShapeDtypeStruct((2, NP, 16), F32),
    compiler_params=_sc_params,
    scratch_types=[
        pltpu.VMEM((4 * GSZ,), jnp.int32),
        pltpu.VMEM((4 * GSZ,), jnp.int32),
        pltpu.VMEM((2 * GSZ, 16), F32),
        pltpu.VMEM_SHARED((NP, 16), F32),
        pltpu.SemaphoreType.DMA,
        pltpu.SemaphoreType.DMA,
        pltpu.SemaphoreType.DMA,
    ],
)
def _edge_pass(g2d, ei, zeros2, out, ridx, cidx, rows_v, acc,
               gsem, ssem, isem):
    c = lax.axis_index("c")
    s = lax.axis_index("s")
    wid = c * 16 + s
    pltpu.sync_copy(zeros2.at[pl.ds(s * SLICE, SLICE)],
                    acc.at[pl.ds(s * SLICE, SLICE)])
    plsc.subcore_barrier()

    base_e = wid * EPT
    last = N_CHUNKS - 1

    def il_descs(t):
        q = t & 3
        return [pltpu.make_async_copy(
                    ei.at[0, pl.ds(base_e + t * GSZ, GSZ)],
                    ridx.at[pl.ds(q * GSZ, GSZ)], isem),
                pltpu.make_async_copy(
                    ei.at[1, pl.ds(base_e + t * GSZ, GSZ)],
                    cidx.at[pl.ds(q * GSZ, GSZ)], isem)]

    def g_descs(t):
        q = t & 3
        p = t & 1
        return [pltpu.make_async_copy(
                    g2d.at[ridx.at[pl.ds(q * GSZ, GSZ)]],
                    rows_v.at[pl.ds(p * GSZ, GSZ)], gsem)]

    def s_descs(t):
        q = t & 3
        p = t & 1
        return [pltpu.make_async_copy(
                    rows_v.at[pl.ds(p * GSZ, GSZ)],
                    acc.at[cidx.at[pl.ds(q * GSZ, GSZ)]], ssem)]

    def fire(descs, add=False):
        for d in descs:
            d.start(add=add)

    def drain(descs):
        for d in descs:
            d.wait()

    fire(il_descs(0))
    fire(il_descs(1))
    fire(il_descs(2))
    drain(il_descs(0))
    fire(g_descs(0))
    drain(il_descs(1))
    fire(g_descs(1))
    drain(g_descs(0))
    fire(s_descs(0), add=True)

    def body(t, carry):
        drain(g_descs(t))
        drain(s_descs(t - 1))
        drain(il_descs(t + 1))
        fire(il_descs(jnp.minimum(t + 2, last)))
        fire(g_descs(t + 1))
        fire(s_descs(t), add=True)
        return carry

    lax.fori_loop(1, N_CHUNKS - 1, body, 0)

    drain(g_descs(last))
    drain(s_descs(last - 1))
    fire(s_descs(last), add=True)
    drain(s_descs(last))
    drain(il_descs(last))

    toff = base_e + N_CHUNKS * GSZ
    pltpu.sync_copy(ei.at[0, pl.ds(toff, TAIL)], ridx.at[pl.ds(0, TAIL)])
    pltpu.sync_copy(ei.at[1, pl.ds(toff, TAIL)], cidx.at[pl.ds(0, TAIL)])
    pltpu.async_copy(g2d.at[ridx.at[pl.ds(0, TAIL)]],
                     rows_v.at[pl.ds(0, TAIL)], gsem).wait()
    tdesc = pltpu.make_async_copy(rows_v.at[pl.ds(0, TAIL)],
                                  acc.at[cidx.at[pl.ds(0, TAIL)]], ssem)
    tdesc.start(add=True)
    tdesc.wait()

    plsc.subcore_barrier()
    pltpu.sync_copy(acc.at[pl.ds(s * SLICE, SLICE)],
                    out.at[c, pl.ds(s * SLICE, SLICE)])


_SELU_SCALE = 1.0507009873554805
_SELU_ALPHA = 1.6732632423543772


def _selu(x):
    return _SELU_SCALE * jnp.where(x > 0, x, _SELU_ALPHA * (jnp.exp(x) - 1.0))


NP8 = NP // 8
BR = NP8 // 8


def _dis128(degp, sel):
    deg8 = degp[0] + degp[1] + 1.0
    return jnp.dot(lax.rsqrt(deg8), sel[...], preferred_element_type=F32)


def _k0_body(degp, sel, xr, w1, b1, wc1, hw1_o, g1_o):
    d = _dis128(degp, sel)
    h0 = jnp.dot(xr[...], w1[...], preferred_element_type=F32) + b1[...]
    hw1 = jnp.dot(h0, wc1[...], preferred_element_type=F32)
    hw1_o[...] = hw1
    g1_o[...] = d * hw1


def _k1_body(degp, sel, raw, hw1, bc1, wc2, hw2_o, g2_o):
    d = _dis128(degp, sel)
    out1 = d * (raw[0] + raw[1]) + d * d * hw1[...] + bc1[...]
    h1 = _selu(out1)
    hw2 = jnp.dot(h1, wc2[...], preferred_element_type=F32)
    hw2_o[...] = hw2
    g2_o[...] = d * hw2


def _k2_body(degp, sel, raw, hw2, bc2, w2, b2, pswap, out_o):
    d = _dis128(degp, sel)
    out2 = d * (raw[0] + raw[1]) + d * d * hw2[...] + bc2[...]
    h2 = _selu(out2)
    z = jnp.dot(h2, w2[...], preferred_element_type=F32) + b2[...]
    zs = jnp.dot(z, pswap[...], preferred_element_type=F32)
    m = jnp.maximum(z, zs)
    lse = m + jnp.log(jnp.exp(z - m) + jnp.exp(zs - m))
    out_o[...] = z - lse


def _full(shape):
    return pl.BlockSpec(shape, lambda i: (0,) * len(shape))


def _rows(f):
    return pl.BlockSpec((BR, f), lambda i: (i, 0))


_DEGSPEC = pl.BlockSpec((2, BR, 8), lambda i: (0, i, 0))

_k0 = pl.pallas_call(
    _k0_body,
    grid=(8,),
    in_specs=[_DEGSPEC, _full((8, 128)), _rows(40), _full((40, 128)),
              _full((1, 128)), _full((128, 128))],
    out_specs=[_rows(128), _rows(128)],
    out_shape=[jax.ShapeDtypeStruct((NP8, 128), F32)] * 2,
)

_k1 = pl.pallas_call(
    _k1_body,
    grid=(8,),
    in_specs=[_DEGSPEC, _full((8, 128)),
              pl.BlockSpec((2, BR, 128), lambda i: (0, i, 0)),
              _rows(128), _full((1, 128)), _full((128, 128))],
    out_specs=[_rows(128), _rows(128)],
    out_shape=[jax.ShapeDtypeStruct((NP8, 128), F32)] * 2,
)

_k2 = pl.pallas_call(
    _k2_body,
    grid=(8,),
    in_specs=[_DEGSPEC, _full((8, 128)),
              pl.BlockSpec((2, BR, 128), lambda i: (0, i, 0)),
              _rows(128), _full((1, 128)), _full((128, 16)),
              _full((1, 16)), _full((16, 16))],
    out_specs=_rows(16),
    out_shape=jax.ShapeDtypeStruct((NP8, 16), F32),
)


def kernel(x, edge_index, W1, b1, Wc1, bc1, Wc2, bc2, W2, b2):
    n = x.shape[0]
    xp = jnp.zeros((NP, x.shape[1]), F32).at[:n].set(x)
    x8 = xp.reshape(NP8, 40)
    zeros1 = jnp.zeros((NP,), F32)
    zeros2 = jnp.zeros((NP, 16), F32)

    eye8 = jnp.eye(8, dtype=F32)
    sel = jnp.kron(eye8, jnp.ones((1, 16), F32))
    w1b = jnp.kron(eye8, W1)
    wc1b = jnp.kron(eye8, Wc1)
    wc2b = jnp.kron(eye8, Wc2)
    w2b = jnp.kron(eye8, W2)
    pswap = jnp.kron(eye8, jnp.array([[0., 1.], [1., 0.]], F32))
    b1t = jnp.tile(b1, 8).reshape(1, 128)
    bc1t = jnp.tile(bc1, 8).reshape(1, 128)
    bc2t = jnp.tile(bc2, 8).reshape(1, 128)
    b2t = jnp.tile(b2, 8).reshape(1, 16)

    deg_p = _deg_pass(edge_index, zeros1)
    deg8 = deg_p.reshape(2, NP8, 8)
    hw1, g1 = _k0(deg8, sel, x8, w1b, b1t, wc1b)
    raw1 = _edge_pass(g1.reshape(NP, 16), edge_index, zeros2)
    hw2, g2 = _k1(deg8, sel, raw1.reshape(2, NP8, 128), hw1, bc1t, wc2b)
    raw2 = _edge_pass(g2.reshape(NP, 16), edge_index, zeros2)
    out8 = _k2(deg8, sel, raw2.reshape(2, NP8, 128), hw2, bc2t, w2b, b2t,
               pswap)
    return out8.reshape(NP, 2)[:n]


# speedup vs baseline: 1.7271x; 1.0008x over previous
import functools

import jax
import jax.numpy as jnp
from jax import lax
from jax.experimental import pallas as pl
from jax.experimental.pallas import tpu as pltpu
from jax.experimental.pallas import tpu_sc as plsc

F32 = jnp.float32

NP = 100352
E_TOT = 3200000
EPT = E_TOT // 32
GSZ = 512
N_CHUNKS = EPT // GSZ
TAIL = EPT - N_CHUNKS * GSZ
SLICE = NP // 16

_mesh = plsc.VectorSubcoreMesh(core_axis_name="c", subcore_axis_name="s")
_sc_params = pltpu.CompilerParams(use_tc_tiling_on_sc=False)


@functools.partial(
    pl.kernel,
    mesh=_mesh,
    out_type=jax.ShapeDtypeStruct((2, NP), F32),
    compiler_params=_sc_params,
    scratch_types=[
        pltpu.VMEM((4 * GSZ,), jnp.int32),
        pltpu.VMEM((GSZ,), F32),
        pltpu.VMEM_SHARED((NP,), F32),
        pltpu.SemaphoreType.DMA,
        pltpu.SemaphoreType.DMA,
    ],
)
def _deg_pass(ei, zeros1, out, cidx, ones_v, acc, ssem, isem):
    c = lax.axis_index("c")
    s = lax.axis_index("s")
    wid = c * 16 + s
    pltpu.sync_copy(zeros1.at[pl.ds(s * SLICE, SLICE)],
                    acc.at[pl.ds(s * SLICE, SLICE)])

    def fill_ones(j, carry):
        ones_v[pl.ds(j * 16, 16)] = jnp.ones((16,), F32)
        return carry

    lax.fori_loop(0, GSZ // 16, fill_ones, 0)
    plsc.subcore_barrier()

    base_e = wid * EPT
    last = N_CHUNKS - 1

    def il_desc(t):
        q = t & 3
        return pltpu.make_async_copy(
            ei.at[1, pl.ds(base_e + t * GSZ, GSZ)],
            cidx.at[pl.ds(q * GSZ, GSZ)], isem)

    def s_descs(t):
        q = t & 3
        return [pltpu.make_async_copy(ones_v,
                                      acc.at[cidx.at[pl.ds(q * GSZ, GSZ)]],
                                      ssem)]

    def s_fire(t):
        for d in s_descs(t):
            d.start(add=True)

    for t in range(4):
        il_desc(t).start()
    for t in range(3):
        il_desc(t).wait()
        s_fire(t)

    def body(t, carry):
        for d in s_descs(t - 3):
            d.wait()
        il_desc(t).wait()
        il_desc(jnp.minimum(t + 1, last)).start()
        s_fire(t)
        return carry

    lax.fori_loop(3, N_CHUNKS, body, 0)
    for t in (N_CHUNKS - 3, N_CHUNKS - 2, N_CHUNKS - 1):
        for d in s_descs(t):
            d.wait()
    il_desc(last).wait()

    pltpu.sync_copy(ei.at[1, pl.ds(base_e + N_CHUNKS * GSZ, TAIL)],
                    cidx.at[pl.ds(0, TAIL)])
    tdesc = pltpu.make_async_copy(ones_v.at[pl.ds(0, TAIL)],
                                  acc.at[cidx.at[pl.ds(0, TAIL)]], ssem)
    tdesc.start(add=True)
    tdesc.wait()

    plsc.subcore_barrier()
    pltpu.sync_copy(acc.at[pl.ds(s * SLICE, SLICE)],
                    out.at[c, pl.ds(s * SLICE, SLICE)])


@functools.partial(
    pl.kernel,
    mesh=_mesh,
    out_type=jax.ShapeDtypeStruct((2, NP, 16), F32),
    compiler_params=_sc_params,
    scratch_types=[
        pltpu.VMEM((4 * GSZ,), jnp.int32),
        pltpu.VMEM((4 * GSZ,), jnp.int32),
        pltpu.VMEM((2 * GSZ, 16), F32),
        pltpu.VMEM_SHARED((NP, 16), F32),
        pltpu.SemaphoreType.DMA,
        pltpu.SemaphoreType.DMA,
        pltpu.SemaphoreType.DMA,
    ],
)
def _edge_pass(g2d, ei, zeros2, out, ridx, cidx, rows_v, acc,
               gsem, ssem, isem):
    c = lax.axis_index("c")
    s = lax.axis_index("s")
    wid = c * 16 + s
    pltpu.sync_copy(zeros2.at[pl.ds(s * SLICE, SLICE)],
                    acc.at[pl.ds(s * SLICE, SLICE)])
    plsc.subcore_barrier()

    base_e = wid * EPT
    last = N_CHUNKS - 1

    def il_descs(t):
        q = t & 3
        return [pltpu.make_async_copy(
                    ei.at[0, pl.ds(base_e + t * GSZ, GSZ)],
                    ridx.at[pl.ds(q * GSZ, GSZ)], isem),
                pltpu.make_async_copy(
                    ei.at[1, pl.ds(base_e + t * GSZ, GSZ)],
                    cidx.at[pl.ds(q * GSZ, GSZ)], isem)]

    def g_descs(t):
        q = t & 3
        p = t & 1
        return [pltpu.make_async_copy(
                    g2d.at[ridx.at[pl.ds(q * GSZ, GSZ)]],
                    rows_v.at[pl.ds(p * GSZ, GSZ)], gsem)]

    def s_descs(t):
        q = t & 3
        p = t & 1
        return [pltpu.make_async_copy(
                    rows_v.at[pl.ds(p * GSZ, GSZ)],
                    acc.at[cidx.at[pl.ds(q * GSZ, GSZ)]], ssem)]

    def fire(descs, add=False):
        for d in descs:
            d.start(add=add)

    def drain(descs):
        for d in descs:
            d.wait()

    fire(il_descs(0))
    fire(il_descs(1))
    fire(il_descs(2))
    drain(il_descs(0))
    fire(g_descs(0))
    drain(il_descs(1))
    fire(g_descs(1))
    drain(g_descs(0))
    fire(s_descs(0), add=True)

    def body(t, carry):
        drain(g_descs(t))
        drain(s_descs(t - 1))
        drain(il_descs(t + 1))
        fire(il_descs(jnp.minimum(t + 2, last)))
        fire(g_descs(t + 1))
        fire(s_descs(t), add=True)
        return carry

    lax.fori_loop(1, N_CHUNKS - 1, body, 0)

    drain(g_descs(last))
    drain(s_descs(last - 1))
    fire(s_descs(last), add=True)
    drain(s_descs(last))
    drain(il_descs(last))

    toff = base_e + N_CHUNKS * GSZ
    pltpu.sync_copy(ei.at[0, pl.ds(toff, TAIL)], ridx.at[pl.ds(0, TAIL)])
    pltpu.sync_copy(ei.at[1, pl.ds(toff, TAIL)], cidx.at[pl.ds(0, TAIL)])
    pltpu.async_copy(g2d.at[ridx.at[pl.ds(0, TAIL)]],
                     rows_v.at[pl.ds(0, TAIL)], gsem).wait()
    tdesc = pltpu.make_async_copy(rows_v.at[pl.ds(0, TAIL)],
                                  acc.at[cidx.at[pl.ds(0, TAIL)]], ssem)
    tdesc.start(add=True)
    tdesc.wait()

    plsc.subcore_barrier()
    pltpu.sync_copy(acc.at[pl.ds(s * SLICE, SLICE)],
                    out.at[c, pl.ds(s * SLICE, SLICE)])


_SELU_SCALE = 1.0507009873554805
_SELU_ALPHA = 1.6732632423543772


def _selu(x):
    return _SELU_SCALE * jnp.where(x > 0, x, _SELU_ALPHA * (jnp.exp(x) - 1.0))


NP8 = NP // 8
BR = NP8 // 8


def _dis128(degp, sel):
    deg8 = degp[0] + degp[1] + 1.0
    return jnp.dot(lax.rsqrt(deg8), sel[...], preferred_element_type=F32)


def _k0_body(degp, sel, xr, w1, b1, wc1, hw1_o, g1_o):
    d = _dis128(degp, sel)
    h0 = jnp.dot(xr[...], w1[...], preferred_element_type=F32) + b1[...]
    hw1 = jnp.dot(h0, wc1[...], preferred_element_type=F32)
    hw1_o[...] = hw1
    g1_o[...] = d * hw1


def _k1_body(degp, sel, raw, hw1, bc1, wc2, hw2_o, g2_o):
    d = _dis128(degp, sel)
    out1 = d * (raw[0] + raw[1]) + d * d * hw1[...] + bc1[...]
    h1 = _selu(out1)
    hw2 = jnp.dot(h1, wc2[...], preferred_element_type=F32)
    hw2_o[...] = hw2
    g2_o[...] = d * hw2


def _k2_body(degp, sel, raw, hw2, bc2, w2, b2, pswap, out_o):
    d = _dis128(degp, sel)
    out2 = d * (raw[0] + raw[1]) + d * d * hw2[...] + bc2[...]
    h2 = _selu(out2)
    z = jnp.dot(h2, w2[...], preferred_element_type=F32) + b2[...]
    zs = jnp.dot(z, pswap[...], preferred_element_type=F32)
    m = jnp.maximum(z, zs)
    lse = m + jnp.log(jnp.exp(z - m) + jnp.exp(zs - m))
    out_o[...] = z - lse


def _full(shape):
    return pl.BlockSpec(shape, lambda i: (0,) * len(shape))


def _rows(f):
    return pl.BlockSpec((BR, f), lambda i: (i, 0))


_DEGSPEC = pl.BlockSpec((2, BR, 8), lambda i: (0, i, 0))

_k0 = pl.pallas_call(
    _k0_body,
    grid=(8,),
    in_specs=[_DEGSPEC, _full((8, 128)), _rows(40), _full((40, 128)),
              _full((1, 128)), _full((128, 128))],
    out_specs=[_rows(128), _rows(128)],
    out_shape=[jax.ShapeDtypeStruct((NP8, 128), F32)] * 2,
)

_k1 = pl.pallas_call(
    _k1_body,
    grid=(8,),
    in_specs=[_DEGSPEC, _full((8, 128)),
              pl.BlockSpec((2, BR, 128), lambda i: (0, i, 0)),
              _rows(128), _full((1, 128)), _full((128, 128))],
    out_specs=[_rows(128), _rows(128)],
    out_shape=[jax.ShapeDtypeStruct((NP8, 128), F32)] * 2,
)

_k2 = pl.pallas_call(
    _k2_body,
    grid=(8,),
    in_specs=[_DEGSPEC, _full((8, 128)),
              pl.BlockSpec((2, BR, 128), lambda i: (0, i, 0)),
              _rows(128), _full((1, 128)), _full((128, 16)),
              _full((1, 16)), _full((16, 16))],
    out_specs=_rows(16),
    out_shape=jax.ShapeDtypeStruct((NP8, 16), F32),
)


def kernel(x, edge_index, W1, b1, Wc1, bc1, Wc2, bc2, W2, b2):
    n = x.shape[0]
    assert n <= NP and edge_index.shape[1] == E_TOT
    xp = jnp.zeros((NP, x.shape[1]), F32).at[:n].set(x)
    x8 = xp.reshape(NP8, 40)
    zeros1 = jnp.zeros((NP,), F32)
    zeros2 = jnp.zeros((NP, 16), F32)

    eye8 = jnp.eye(8, dtype=F32)
    sel = jnp.kron(eye8, jnp.ones((1, 16), F32))
    w1b = jnp.kron(eye8, W1)
    wc1b = jnp.kron(eye8, Wc1)
    wc2b = jnp.kron(eye8, Wc2)
    w2b = jnp.kron(eye8, W2)
    pswap = jnp.kron(eye8, jnp.array([[0., 1.], [1., 0.]], F32))
    b1t = jnp.tile(b1, 8).reshape(1, 128)
    bc1t = jnp.tile(bc1, 8).reshape(1, 128)
    bc2t = jnp.tile(bc2, 8).reshape(1, 128)
    b2t = jnp.tile(b2, 8).reshape(1, 16)

    deg_p = _deg_pass(edge_index, zeros1)
    deg8 = deg_p.reshape(2, NP8, 8)
    hw1, g1 = _k0(deg8, sel, x8, w1b, b1t, wc1b)
    raw1 = _edge_pass(g1.reshape(NP, 16), edge_index, zeros2)
    hw2, g2 = _k1(deg8, sel, raw1.reshape(2, NP8, 128), hw1, bc1t, wc2b)
    raw2 = _edge_pass(g2.reshape(NP, 16), edge_index, zeros2)
    out8 = _k2(deg8, sel, raw2.reshape(2, NP8, 128), hw2, bc2t, w2b, b2t,
               pswap)
    return out8.reshape(NP, 2)[:n]
